# baseline (speedup 1.0000x reference)
import jax
import jax.numpy as jnp
from jax import lax
from jax.experimental import pallas as pl
from jax.experimental.pallas import tpu as pltpu
from jax.experimental.pallas import tpu_sc as plsc

N = 10000
E = 320000
D = 128

NC = 2
NS = 16
NW = NC * NS
EP = 10240
B = 128
CHS = 8
SL = 10
NP = 10240
RPT = NP // NS


def _sc_body(x_hbm, meta_hbm, agg_hbm, ws_hbm,
             meta_v, gbuf, gbuf2, ws_v, acc_sh, gsem):
    c = lax.axis_index("c")
    s = lax.axis_index("s")
    wid = c * NS + s

    def zero_gbuf(i, _):
        for k in range(D // 16):
            gbuf[i, pl.ds(16 * k, 16)] = jnp.zeros((16,), jnp.float32)
        return _
    lax.fori_loop(0, B, zero_gbuf, None)

    def zero_ws(i, _):
        ws_v[pl.ds(i * 16, 16)] = jnp.zeros((16,), jnp.float32)
        return _
    lax.fori_loop(0, NP // 16, zero_ws, None)

    for r in range(RPT // B):
        pltpu.sync_copy(gbuf, acc_sh.at[pl.ds(s * RPT + r * B, B)])
    plsc.subcore_barrier()

    gbufs = (gbuf, gbuf2)

    def compute_chunk(sl, j, gb):
        for g in range(B // 16):
            wv = plsc.bitcast(meta_v[3 * j + 2, pl.ds(16 * g, 16)], jnp.float32)
            iv = meta_v[3 * j + 1, pl.ds(16 * g, 16)]
            plsc.addupdate_scatter(ws_v, [iv], wv)
            for l in range(16):
                e = 16 * g + l
                w = wv[l]
                for k in range(D // 16):
                    gb[e, pl.ds(16 * k, 16)] = (
                        gb[e, pl.ds(16 * k, 16)] * w)
        pltpu.sync_copy(gb, acc_sh.at[meta_v.at[3 * j + 1]], add=True)

    def slab(sl, carry):
        pltpu.sync_copy(meta_hbm.at[wid, sl], meta_v)
        pltpu.make_async_copy(
            x_hbm.at[meta_v.at[0]], gbufs[0], gsem).start()

        def pair(p, carry2):
            for u in range(2):
                j = 2 * p + u
                pltpu.make_async_copy(
                    x_hbm.at[meta_v.at[3 * j]], gbufs[u], gsem).wait()

                @pl.when(j + 1 < CHS)
                def _():
                    pltpu.make_async_copy(
                        x_hbm.at[meta_v.at[3 * (j + 1)]],
                        gbufs[1 - u], gsem).start()

                compute_chunk(sl, j, gbufs[u])
            return carry2
        lax.fori_loop(0, CHS // 2, pair, 0)
        return carry
    lax.fori_loop(0, SL, slab, 0)

    plsc.subcore_barrier()

    pltpu.sync_copy(acc_sh.at[pl.ds(s * RPT, RPT)],
                    agg_hbm.at[c, pl.ds(s * RPT, RPT)])
    pltpu.sync_copy(ws_v, ws_hbm.at[wid])


def _sc_aggregate(x, meta):
    mesh = plsc.VectorSubcoreMesh(core_axis_name="c", subcore_axis_name="s")
    return pl.kernel(
        _sc_body,
        out_type=(
            jax.ShapeDtypeStruct((NC, NP, D), jnp.float32),
            jax.ShapeDtypeStruct((NW, NP), jnp.float32),
        ),
        mesh=mesh,
        compiler_params=pltpu.CompilerParams(needs_layout_passes=False),
        scratch_types=[
            pltpu.VMEM((CHS * 3, B), jnp.int32),
            pltpu.VMEM((B, D), jnp.float32),
            pltpu.VMEM((B, D), jnp.float32),
            pltpu.VMEM((NP,), jnp.float32),
            pltpu.VMEM_SHARED((NP, D), jnp.float32),
            pltpu.SemaphoreType.DMA,
        ],
    )(x, meta)


def _tc_self_body(x_ref, wsT_ref, b_ref, o_ref):
    o_ref[...] = jnp.dot(x_ref[...], wsT_ref[...],
                         preferred_element_type=jnp.float32) + b_ref[...]


def _tc_rest_body(h0_ref, p_ref, w_ref, wnT_ref, o_ref):
    agg = p_ref[0] + p_ref[1]
    wsum = jnp.sum(w_ref[...], axis=0)
    neigh = agg / jnp.maximum(wsum, 1e-8)[:, None]
    h = h0_ref[...] + jnp.dot(neigh, wnT_ref[...],
                              preferred_element_type=jnp.float32)
    nrm = jnp.sqrt(jnp.sum(h * h, axis=1, keepdims=True))
    o_ref[...] = h / jnp.maximum(nrm, 1e-12)


def _tc_self(x, wsT, bias2d):
    R = 1024
    grid = (pl.cdiv(N, R),)
    return pl.pallas_call(
        _tc_self_body,
        grid=grid,
        in_specs=[
            pl.BlockSpec((R, D), lambda i: (i, 0)),
            pl.BlockSpec((D, D), lambda i: (0, 0)),
            pl.BlockSpec((1, D), lambda i: (0, 0)),
        ],
        out_specs=pl.BlockSpec((R, D), lambda i: (i, 0)),
        out_shape=jax.ShapeDtypeStruct((N, D), jnp.float32),
    )(x, wsT, bias2d)


def _tc_rest(h0, partials, wsums, wnT):
    R = 1024
    grid = (pl.cdiv(N, R),)
    return pl.pallas_call(
        _tc_rest_body,
        grid=grid,
        in_specs=[
            pl.BlockSpec((R, D), lambda i: (i, 0)),
            pl.BlockSpec((NC, R, D), lambda i: (0, i, 0)),
            pl.BlockSpec((NW, R), lambda i: (0, i)),
            pl.BlockSpec((D, D), lambda i: (0, 0)),
        ],
        out_specs=pl.BlockSpec((R, D), lambda i: (i, 0)),
        out_shape=jax.ShapeDtypeStruct((N, D), jnp.float32),
    )(h0, partials, wsums, wnT)


@jax.jit
def kernel(x, edge_index, edge_weight, W_self, W_neigh, bias):
    pad = NW * EP - E
    spread = (jnp.arange(pad, dtype=jnp.int32) * 131) % N
    row = jnp.concatenate(
        [edge_index[0], spread]).reshape(NW, SL, CHS, B)
    col = jnp.concatenate(
        [edge_index[1], spread]).reshape(NW, SL, CHS, B)
    ewb = lax.bitcast_convert_type(
        jnp.concatenate([edge_weight, jnp.zeros((pad,), jnp.float32)]),
        jnp.int32).reshape(NW, SL, CHS, B)
    meta = jnp.stack([row, col, ewb], axis=3).reshape(NW, SL, CHS * 3, B)
    partials, wsums = _sc_aggregate(x, meta)
    h0 = _tc_self(x, W_self.T, bias.reshape(1, D))
    return _tc_rest(h0, partials, wsums, W_neigh.T)


# speedup vs baseline: 1.1595x; 1.1595x over previous
import jax
import jax.numpy as jnp
from jax import lax
from jax.experimental import pallas as pl
from jax.experimental.pallas import tpu as pltpu
from jax.experimental.pallas import tpu_sc as plsc

N = 10000
E = 320000
D = 128

NC = 2
NS = 16
NW = NC * NS
EP = 10240
B = 80
CHS = 8
SL = 16
NP = 10240
RPT = NP // NS


def _sc_body(x_hbm, meta_hbm, agg_hbm, ws_hbm,
             meta_v, gbuf, gbuf2, pbuf, ws_v, acc_sh, gsem):
    c = lax.axis_index("c")
    s = lax.axis_index("s")
    wid = c * NS + s

    def zero_pbuf(i, _):
        for k in range(D // 16):
            pbuf[i, pl.ds(16 * k, 16)] = jnp.zeros((16,), jnp.float32)
        return _
    lax.fori_loop(0, B, zero_pbuf, None)

    def zero_ws(i, _):
        ws_v[pl.ds(i * 16, 16)] = jnp.zeros((16,), jnp.float32)
        return _
    lax.fori_loop(0, NP // 16, zero_ws, None)

    for r in range(RPT // B):
        pltpu.sync_copy(pbuf, acc_sh.at[pl.ds(s * RPT + r * B, B)])
    plsc.subcore_barrier()

    gbufs = (gbuf, gbuf2)

    def compute_chunk(sl, j, gb, pb):
        for g in range(B // 16):
            wv = plsc.bitcast(meta_v[3 * j + 2, pl.ds(16 * g, 16)], jnp.float32)
            iv = meta_v[3 * j + 1, pl.ds(16 * g, 16)]
            plsc.addupdate_scatter(ws_v, [iv], wv)
            for l in range(16):
                e = 16 * g + l
                w = wv[l]
                for k in range(D // 16):
                    pb[e, pl.ds(16 * k, 16)] = (
                        gb[e, pl.ds(16 * k, 16)] * w)
        pltpu.sync_copy(pb, acc_sh.at[meta_v.at[3 * j + 1]], add=True)

    def slab(sl, carry):
        pltpu.sync_copy(meta_hbm.at[wid, sl], meta_v)
        pltpu.make_async_copy(
            x_hbm.at[meta_v.at[0]], gbufs[0], gsem).start()

        def pair(p, carry2):
            for u in range(2):
                j = 2 * p + u
                pltpu.make_async_copy(
                    x_hbm.at[meta_v.at[3 * j]], gbufs[u], gsem).wait()

                @pl.when(j + 1 < CHS)
                def _():
                    pltpu.make_async_copy(
                        x_hbm.at[meta_v.at[3 * (j + 1)]],
                        gbufs[1 - u], gsem).start()

                compute_chunk(sl, j, gbufs[u], pbuf)
            return carry2
        lax.fori_loop(0, CHS // 2, pair, 0)
        return carry
    lax.fori_loop(0, SL, slab, 0)

    plsc.subcore_barrier()

    pltpu.sync_copy(acc_sh.at[pl.ds(s * RPT, RPT)],
                    agg_hbm.at[c, pl.ds(s * RPT, RPT)])
    pltpu.sync_copy(ws_v, ws_hbm.at[wid])


def _sc_aggregate(x, meta):
    mesh = plsc.VectorSubcoreMesh(core_axis_name="c", subcore_axis_name="s")
    return pl.kernel(
        _sc_body,
        out_type=(
            jax.ShapeDtypeStruct((NC, NP, D), jnp.float32),
            jax.ShapeDtypeStruct((NW, NP), jnp.float32),
        ),
        mesh=mesh,
        compiler_params=pltpu.CompilerParams(needs_layout_passes=False),
        scratch_types=[
            pltpu.VMEM((CHS * 3, B), jnp.int32),
            pltpu.VMEM((B, D), jnp.float32),
            pltpu.VMEM((B, D), jnp.float32),
            pltpu.VMEM((B, D), jnp.float32),
            pltpu.VMEM((NP,), jnp.float32),
            pltpu.VMEM_SHARED((NP, D), jnp.float32),
            pltpu.SemaphoreType.DMA,
        ],
    )(x, meta)


def _tc_self_body(x_ref, wsT_ref, b_ref, o_ref):
    o_ref[...] = jnp.dot(x_ref[...], wsT_ref[...],
                         preferred_element_type=jnp.float32) + b_ref[...]


def _tc_rest_body(h0_ref, p_ref, w_ref, wnT_ref, o_ref):
    agg = p_ref[0] + p_ref[1]
    wsum = jnp.sum(w_ref[...], axis=0)
    neigh = agg / jnp.maximum(wsum, 1e-8)[:, None]
    h = h0_ref[...] + jnp.dot(neigh, wnT_ref[...],
                              preferred_element_type=jnp.float32)
    nrm = jnp.sqrt(jnp.sum(h * h, axis=1, keepdims=True))
    o_ref[...] = h / jnp.maximum(nrm, 1e-12)


def _tc_self(x, wsT, bias2d):
    R = 1024
    grid = (pl.cdiv(N, R),)
    return pl.pallas_call(
        _tc_self_body,
        grid=grid,
        in_specs=[
            pl.BlockSpec((R, D), lambda i: (i, 0)),
            pl.BlockSpec((D, D), lambda i: (0, 0)),
            pl.BlockSpec((1, D), lambda i: (0, 0)),
        ],
        out_specs=pl.BlockSpec((R, D), lambda i: (i, 0)),
        out_shape=jax.ShapeDtypeStruct((N, D), jnp.float32),
    )(x, wsT, bias2d)


def _tc_rest(h0, partials, wsums, wnT):
    R = 1024
    grid = (pl.cdiv(N, R),)
    return pl.pallas_call(
        _tc_rest_body,
        grid=grid,
        in_specs=[
            pl.BlockSpec((R, D), lambda i: (i, 0)),
            pl.BlockSpec((NC, R, D), lambda i: (0, i, 0)),
            pl.BlockSpec((NW, R), lambda i: (0, i)),
            pl.BlockSpec((D, D), lambda i: (0, 0)),
        ],
        out_specs=pl.BlockSpec((R, D), lambda i: (i, 0)),
        out_shape=jax.ShapeDtypeStruct((N, D), jnp.float32),
    )(h0, partials, wsums, wnT)


@jax.jit
def kernel(x, edge_index, edge_weight, W_self, W_neigh, bias):
    pad = NW * EP - E
    spread = (jnp.arange(pad, dtype=jnp.int32) * 131) % N
    row = jnp.concatenate(
        [edge_index[0], spread]).reshape(NW, SL, CHS, B)
    col = jnp.concatenate(
        [edge_index[1], spread]).reshape(NW, SL, CHS, B)
    ewb = lax.bitcast_convert_type(
        jnp.concatenate([edge_weight, jnp.zeros((pad,), jnp.float32)]),
        jnp.int32).reshape(NW, SL, CHS, B)
    meta = jnp.stack([row, col, ewb], axis=3).reshape(NW, SL, CHS * 3, B)
    partials, wsums = _sc_aggregate(x, meta)
    h0 = _tc_self(x, W_self.T, bias.reshape(1, D))
    return _tc_rest(h0, partials, wsums, W_neigh.T)
